# baseline (speedup 1.0000x reference)
import functools

import jax
import jax.numpy as jnp
from jax import lax
from jax.experimental import pallas as pl
from jax.experimental.pallas import tpu as pltpu
from jax.experimental.pallas import tpu_sc as plsc

BATCH = 16384
EMBED = 64
NWORKERS = 32
BPW = BATCH // NWORKERS
IDX_CHUNK = 128
NCHUNK = BPW // IDX_CHUNK


def _sc_gather(uids2d, iids2d, user_table, item_table):
    mesh = plsc.VectorSubcoreMesh(core_axis_name="c", subcore_axis_name="s")

    @functools.partial(
        pl.kernel,
        mesh=mesh,
        out_type=(
            jax.ShapeDtypeStruct((BATCH, EMBED), jnp.float32),
            jax.ShapeDtypeStruct((BATCH, EMBED), jnp.float32),
        ),
        scratch_types=[
            pltpu.VMEM((NCHUNK, IDX_CHUNK), jnp.int32),
            pltpu.VMEM((NCHUNK, IDX_CHUNK), jnp.int32),
            pltpu.VMEM((BPW, EMBED), jnp.float32),
            pltpu.VMEM((BPW, EMBED), jnp.float32),
            pltpu.SemaphoreType.DMA,
        ],
        compiler_params=pltpu.CompilerParams(use_tc_tiling_on_sc=False),
    )
    def gather_kernel(uids_hbm, iids_hbm, ut_hbm, it_hbm, uout_hbm, iout_hbm,
                      uidx_v, iidx_v, urows_v, irows_v, sem):
        wid = lax.axis_index("s") * 2 + lax.axis_index("c")
        base = wid * BPW
        idx_row0 = wid * NCHUNK
        pltpu.sync_copy(uids_hbm.at[pl.ds(idx_row0, NCHUNK), :], uidx_v)
        pltpu.sync_copy(iids_hbm.at[pl.ds(idx_row0, NCHUNK), :], iidx_v)
        copies = []
        for j in range(NCHUNK):
            dst = urows_v.at[pl.ds(j * IDX_CHUNK, IDX_CHUNK), :]
            copies.append(pltpu.async_copy(ut_hbm.at[uidx_v.at[j]], dst, sem))
            dst = irows_v.at[pl.ds(j * IDX_CHUNK, IDX_CHUNK), :]
            copies.append(pltpu.async_copy(it_hbm.at[iidx_v.at[j]], dst, sem))
        for c in copies:
            c.wait()
        pltpu.sync_copy(urows_v, uout_hbm.at[pl.ds(base, BPW), :])
        pltpu.sync_copy(irows_v, iout_hbm.at[pl.ds(base, BPW), :])

    return gather_kernel(uids2d, iids2d, user_table, item_table)


def _mlp_body(ue_ref, ie_ref, w1a_ref, w1b_ref, b1_ref, w2_ref, b2_ref,
              w3_ref, b3_ref, out_ref):
    h1 = jnp.dot(ue_ref[...], w1a_ref[...], preferred_element_type=jnp.float32)
    h1 += jnp.dot(ie_ref[...], w1b_ref[...], preferred_element_type=jnp.float32)
    h1 = jnp.maximum(h1 + b1_ref[...], 0.0)
    h2 = jnp.dot(h1, w2_ref[...], preferred_element_type=jnp.float32)
    h2 = jnp.maximum(h2 + b2_ref[...], 0.0)
    logit = jnp.dot(h2, w3_ref[...], preferred_element_type=jnp.float32)
    logit = logit + b3_ref[...]
    out_ref[...] = 1.0 / (1.0 + jnp.exp(-logit))


def _mlp(user_emb, item_emb, w1a, w1b, b1, w2, b2, w3, b3, interpret=False):
    BM = 2048
    grid = (BATCH // BM,)

    def full(shape):
        return pl.BlockSpec(shape, lambda i: (0, 0))

    return pl.pallas_call(
        _mlp_body,
        grid=grid,
        in_specs=[
            pl.BlockSpec((BM, EMBED), lambda i: (i, 0)),
            pl.BlockSpec((BM, EMBED), lambda i: (i, 0)),
            full((EMBED, 128)),
            full((EMBED, 128)),
            full((1, 128)),
            full((128, EMBED)),
            full((1, EMBED)),
            full((EMBED, 1)),
            full((1, 1)),
        ],
        out_specs=pl.BlockSpec((BM, 1), lambda i: (i, 0)),
        out_shape=jax.ShapeDtypeStruct((BATCH, 1), jnp.float32),
        interpret=interpret,
    )(user_emb, item_emb, w1a, w1b, b1, w2, b2, w3, b3)


def kernel(user_ids, item_ids, user_table, item_table, W1, b1, W2, b2, W3, b3):
    uids2d = user_ids.astype(jnp.int32).reshape(BATCH // IDX_CHUNK, IDX_CHUNK)
    iids2d = item_ids.astype(jnp.int32).reshape(BATCH // IDX_CHUNK, IDX_CHUNK)
    user_emb, item_emb = _sc_gather(uids2d, iids2d, user_table, item_table)
    w1a = W1[:, :EMBED].T
    w1b = W1[:, EMBED:].T
    return _mlp(user_emb, item_emb, w1a, w1b, b1.reshape(1, 128),
                W2.T, b2.reshape(1, EMBED), W3.T, b3.reshape(1, 1))
